# Initial kernel scaffold; baseline (speedup 1.0000x reference)
import jax
import jax.numpy as jnp
from jax.experimental import pallas as pl


def kernel(xyz1, xyz2, points1, points2, W0, b0, W1, b1):
    raise NotImplementedError("write your pallas kernel here")


# speedup vs baseline: 13.7409x; 13.7409x over previous
import functools

import jax
import jax.numpy as jnp
from jax import lax
from jax.experimental import pallas as pl
from jax.experimental.pallas import tpu as pltpu
from jax.experimental.pallas import tpu_sc as plsc

_NC = 2
_NS = 16
_NW = _NC * _NS
_L = 16


def _nn3_kernel(n2, blk, x1_ref, x2t_ref, idx_ref, w_ref):
    b = pl.program_id(0)
    x1 = x1_ref[0]
    x2t = x2t_ref[0]
    d2 = None
    for c in range(3):
        diff = x1[:, c:c + 1] - x2t[c:c + 1, :]
        d2 = diff * diff if d2 is None else d2 + diff * diff
    j = lax.broadcasted_iota(jnp.int32, d2.shape, 1)
    idxs, invs = [], []
    for k in range(3):
        m = jnp.min(d2, axis=1, keepdims=True)
        ik = jnp.min(jnp.where(d2 == m, j, n2), axis=1, keepdims=True)
        idxs.append(ik)
        invs.append(1.0 / jnp.maximum(m, 1e-10))
        if k < 2:
            d2 = jnp.where(j == ik, jnp.inf, d2)
    norm = invs[0] + invs[1] + invs[2]
    idx_ref[0] = jnp.concatenate([ik + b * n2 for ik in idxs], axis=1)
    w_ref[0] = jnp.concatenate(
        [jnp.broadcast_to(inv / norm, (blk, _L)) for inv in invs], axis=1)


def _mlp_kernel(it_ref, p1_ref, w0a_ref, w0b_ref, b0_ref, w1_ref, b1_ref,
                o_ref):
    h = jnp.dot(it_ref[...], w0a_ref[...], preferred_element_type=jnp.float32)
    h = h + jnp.dot(p1_ref[...], w0b_ref[...],
                    preferred_element_type=jnp.float32)
    h = jnp.maximum(h + b0_ref[...], 0.0)
    o = jnp.dot(h, w1_ref[...], preferred_element_type=jnp.float32)
    o_ref[...] = jnp.maximum(o + b1_ref[...], 0.0)


def kernel(xyz1, xyz2, points1, points2, W0, b0, W1, b1):
    B, N1, _ = xyz1.shape
    N2 = xyz2.shape[1]
    C1 = points1.shape[2]
    C2 = points2.shape[2]
    H = W0.shape[1]
    H2 = W1.shape[1]
    Q = B * N1

    BLK = 512
    nn3 = pl.pallas_call(
        functools.partial(_nn3_kernel, N2, BLK),
        grid=(B, N1 // BLK),
        in_specs=[
            pl.BlockSpec((1, BLK, 3), lambda b, n: (b, n, 0)),
            pl.BlockSpec((1, 3, N2), lambda b, n: (b, 0, 0)),
        ],
        out_specs=[
            pl.BlockSpec((1, BLK, 3), lambda b, n: (b, n, 0)),
            pl.BlockSpec((1, BLK, 3 * _L), lambda b, n: (b, n, 0)),
        ],
        out_shape=[
            jax.ShapeDtypeStruct((B, N1, 3), jnp.int32),
            jax.ShapeDtypeStruct((B, N1, 3 * _L), jnp.float32),
        ],
    )
    idx3, w3 = nn3(xyz1, xyz2.transpose(0, 2, 1))

    QPW = Q // _NW
    CH = 64
    NCH = QPW // CH
    mesh = plsc.VectorSubcoreMesh(core_axis_name="c", subcore_axis_name="s")

    @functools.partial(
        pl.kernel,
        mesh=mesh,
        out_type=jax.ShapeDtypeStruct((Q, C2), jnp.float32),
        scratch_types=[
            pltpu.VMEM((CH * 3,), jnp.int32),
            pltpu.VMEM((CH * 3 * _L,), jnp.float32),
            pltpu.VMEM((CH * 3, C2), jnp.float32),
            pltpu.VMEM((CH, C2), jnp.float32),
            pltpu.SemaphoreType.DMA,
        ],
    )
    def sc_interp(p2_hbm, idx_hbm, w_hbm, out_hbm, idx_v, w_v, rows_v, out_v,
                  sem):
        wid = lax.axis_index("s") * _NC + lax.axis_index("c")
        qw = wid * QPW
        nf = C2 // _L
        for ci in range(NCH):
            qb = qw + ci * CH
            pltpu.sync_copy(idx_hbm.at[pl.ds(qb * 3, CH * 3)], idx_v)
            pltpu.sync_copy(w_hbm.at[pl.ds(qb * 3 * _L, CH * 3 * _L)], w_v)
            pltpu.async_copy(p2_hbm.at[idx_v], rows_v, sem).wait()

            def body(i, _):
                w0v = w_v[pl.ds(i * 3 * _L, _L)]
                w1v = w_v[pl.ds(i * 3 * _L + _L, _L)]
                w2v = w_v[pl.ds(i * 3 * _L + 2 * _L, _L)]
                for f in range(nf):
                    sl = pl.ds(f * _L, _L)
                    acc = w0v * rows_v[3 * i, sl]
                    acc = acc + w1v * rows_v[3 * i + 1, sl]
                    acc = acc + w2v * rows_v[3 * i + 2, sl]
                    out_v[i, sl] = acc
                return 0

            lax.fori_loop(0, CH, body, 0)
            pltpu.sync_copy(out_v, out_hbm.at[pl.ds(qb, CH)])

    interp = sc_interp(points2.reshape(B * N2, C2),
                       idx3.reshape(Q * 3),
                       w3.reshape(Q * 3 * _L))

    MB = 1024
    mlp = pl.pallas_call(
        _mlp_kernel,
        grid=(Q // MB,),
        in_specs=[
            pl.BlockSpec((MB, C2), lambda r: (r, 0)),
            pl.BlockSpec((MB, C1), lambda r: (r, 0)),
            pl.BlockSpec((C2, H), lambda r: (0, 0)),
            pl.BlockSpec((C1, H), lambda r: (0, 0)),
            pl.BlockSpec((1, H), lambda r: (0, 0)),
            pl.BlockSpec((H, H2), lambda r: (0, 0)),
            pl.BlockSpec((1, H2), lambda r: (0, 0)),
        ],
        out_specs=pl.BlockSpec((MB, H2), lambda r: (r, 0)),
        out_shape=jax.ShapeDtypeStruct((Q, H2), jnp.float32),
    )
    out = mlp(interp, points1.reshape(Q, C1), W0[:C2], W0[C2:],
              b0.reshape(1, H), W1, b1.reshape(1, H2))
    return out.reshape(B, N1, H2)


# speedup vs baseline: 15.8028x; 1.1501x over previous
import functools

import jax
import jax.numpy as jnp
from jax import lax
from jax.experimental import pallas as pl
from jax.experimental.pallas import tpu as pltpu
from jax.experimental.pallas import tpu_sc as plsc

_NC = 2
_NS = 16
_NW = _NC * _NS
_L = 16


def _nn3_kernel(n2, blk, x1_ref, x2t_ref, idx_ref, w_ref):
    b = pl.program_id(0)
    x1 = x1_ref[0]
    x2t = x2t_ref[0]
    d2 = None
    for c in range(3):
        diff = x1[:, c:c + 1] - x2t[c:c + 1, :]
        d2 = diff * diff if d2 is None else d2 + diff * diff
    j = lax.broadcasted_iota(jnp.int32, d2.shape, 1)
    idxs, invs = [], []
    for k in range(3):
        m = jnp.min(d2, axis=1, keepdims=True)
        ik = jnp.min(jnp.where(d2 == m, j, n2), axis=1, keepdims=True)
        idxs.append(ik)
        invs.append(1.0 / jnp.maximum(m, 1e-10))
        if k < 2:
            d2 = jnp.where(j == ik, jnp.inf, d2)
    norm = invs[0] + invs[1] + invs[2]
    idx_ref[0] = jnp.concatenate([ik + b * n2 for ik in idxs], axis=1)
    w_ref[0] = jnp.concatenate(
        [jnp.broadcast_to(inv / norm, (blk, _L)) for inv in invs], axis=1)


def _mlp_kernel(it_ref, p1_ref, w0a_ref, w0b_ref, b0_ref, w1_ref, b1_ref,
                o_ref):
    h = jnp.dot(it_ref[...], w0a_ref[...], preferred_element_type=jnp.float32)
    h = h + jnp.dot(p1_ref[...], w0b_ref[...],
                    preferred_element_type=jnp.float32)
    h = jnp.maximum(h + b0_ref[...], 0.0)
    o = jnp.dot(h, w1_ref[...], preferred_element_type=jnp.float32)
    o_ref[...] = jnp.maximum(o + b1_ref[...], 0.0)


def kernel(xyz1, xyz2, points1, points2, W0, b0, W1, b1):
    B, N1, _ = xyz1.shape
    N2 = xyz2.shape[1]
    C1 = points1.shape[2]
    C2 = points2.shape[2]
    H = W0.shape[1]
    H2 = W1.shape[1]
    Q = B * N1

    BLK = 512
    nn3 = pl.pallas_call(
        functools.partial(_nn3_kernel, N2, BLK),
        grid=(B, N1 // BLK),
        in_specs=[
            pl.BlockSpec((1, BLK, 3), lambda b, n: (b, n, 0)),
            pl.BlockSpec((1, 3, N2), lambda b, n: (b, 0, 0)),
        ],
        out_specs=[
            pl.BlockSpec((1, BLK, 3), lambda b, n: (b, n, 0)),
            pl.BlockSpec((1, BLK, 3 * _L), lambda b, n: (b, n, 0)),
        ],
        out_shape=[
            jax.ShapeDtypeStruct((B, N1, 3), jnp.int32),
            jax.ShapeDtypeStruct((B, N1, 3 * _L), jnp.float32),
        ],
    )
    idx3, w3 = nn3(xyz1, xyz2.transpose(0, 2, 1))

    QPW = Q // _NW
    CH = 32
    NCH = QPW // CH
    nf = C2 // _L
    mesh = plsc.VectorSubcoreMesh(core_axis_name="c", subcore_axis_name="s")

    @functools.partial(
        pl.kernel,
        mesh=mesh,
        out_type=jax.ShapeDtypeStruct((Q, C2), jnp.float32),
        scratch_types=[
            pltpu.VMEM((QPW * 3,), jnp.int32),
            pltpu.VMEM((CH * 3 * _L,), jnp.float32),
            pltpu.VMEM((CH * 3 * _L,), jnp.float32),
            pltpu.VMEM((CH * 3, C2), jnp.float32),
            pltpu.VMEM((CH * 3, C2), jnp.float32),
            pltpu.VMEM((CH, C2), jnp.float32),
            pltpu.VMEM((CH, C2), jnp.float32),
            pltpu.SemaphoreType.DMA,
            pltpu.SemaphoreType.DMA,
            pltpu.SemaphoreType.DMA,
            pltpu.SemaphoreType.DMA,
            pltpu.SemaphoreType.DMA,
            pltpu.SemaphoreType.DMA,
        ],
    )
    def sc_interp(p2_hbm, idx_hbm, w_hbm, out_hbm, idx_v, w_b0, w_b1, r_b0,
                  r_b1, o_b0, o_b1, sg0, sg1, sw0, sw1, so0, so1):
        wid = lax.axis_index("s") * _NC + lax.axis_index("c")
        qw = wid * QPW
        w_b, r_b, o_b = [w_b0, w_b1], [r_b0, r_b1], [o_b0, o_b1]
        sg, sw, so = [sg0, sg1], [sw0, sw1], [so0, so1]
        gd, wd, od = [None, None], [None, None], [None, None]

        pltpu.sync_copy(idx_hbm.at[pl.ds(qw * 3, QPW * 3)], idx_v)

        def start(ci):
            buf = ci % 2
            gd[buf] = pltpu.async_copy(
                p2_hbm.at[idx_v.at[pl.ds(ci * CH * 3, CH * 3)]], r_b[buf],
                sg[buf])
            wd[buf] = pltpu.async_copy(
                w_hbm.at[pl.ds((qw + ci * CH) * 3 * _L, CH * 3 * _L)],
                w_b[buf], sw[buf])

        start(0)
        for ci in range(NCH):
            buf = ci % 2
            if ci + 1 < NCH:
                start(ci + 1)
            gd[buf].wait()
            wd[buf].wait()
            if od[buf] is not None:
                od[buf].wait()
            rows, wv, ov = r_b[buf], w_b[buf], o_b[buf]

            def body(i, _):
                for u in range(2):
                    q = 2 * i + u
                    base = q * 3 * _L
                    w0v = wv[pl.ds(base, _L)]
                    w1v = wv[pl.ds(base + _L, _L)]
                    w2v = wv[pl.ds(base + 2 * _L, _L)]
                    for f in range(nf):
                        sl = pl.ds(f * _L, _L)
                        acc = w0v * rows[3 * q, sl]
                        acc = acc + w1v * rows[3 * q + 1, sl]
                        acc = acc + w2v * rows[3 * q + 2, sl]
                        ov[q, sl] = acc
                return 0

            lax.fori_loop(0, CH // 2, body, 0)
            od[buf] = pltpu.async_copy(
                ov, out_hbm.at[pl.ds(qw + ci * CH, CH)], so[buf])
        od[0].wait()
        od[1].wait()

    interp = sc_interp(points2.reshape(B * N2, C2),
                       idx3.reshape(Q * 3),
                       w3.reshape(Q * 3 * _L))

    MB = 1024
    mlp = pl.pallas_call(
        _mlp_kernel,
        grid=(Q // MB,),
        in_specs=[
            pl.BlockSpec((MB, C2), lambda r: (r, 0)),
            pl.BlockSpec((MB, C1), lambda r: (r, 0)),
            pl.BlockSpec((C2, H), lambda r: (0, 0)),
            pl.BlockSpec((C1, H), lambda r: (0, 0)),
            pl.BlockSpec((1, H), lambda r: (0, 0)),
            pl.BlockSpec((H, H2), lambda r: (0, 0)),
            pl.BlockSpec((1, H2), lambda r: (0, 0)),
        ],
        out_specs=pl.BlockSpec((MB, H2), lambda r: (r, 0)),
        out_shape=jax.ShapeDtypeStruct((Q, H2), jnp.float32),
    )
    out = mlp(interp, points1.reshape(Q, C1), W0[:C2], W0[C2:],
              b0.reshape(1, H), W1, b1.reshape(1, H2))
    return out.reshape(B, N1, H2)


# speedup vs baseline: 17.4693x; 1.1055x over previous
import functools

import jax
import jax.numpy as jnp
from jax import lax
from jax.experimental import pallas as pl
from jax.experimental.pallas import tpu as pltpu
from jax.experimental.pallas import tpu_sc as plsc

_NC = 2
_NS = 16
_NW = _NC * _NS
_L = 16


def _nn3_kernel(n2, blk, x1_ref, x2t_ref, idx_ref, w_ref):
    b = pl.program_id(0)
    x1 = x1_ref[0]
    x2t = x2t_ref[0]
    d2 = None
    for c in range(3):
        diff = x1[:, c:c + 1] - x2t[c:c + 1, :]
        d2 = diff * diff if d2 is None else d2 + diff * diff
    j = lax.broadcasted_iota(jnp.int32, d2.shape, 1)
    idxs, invs = [], []
    for k in range(3):
        m = jnp.min(d2, axis=1, keepdims=True)
        ik = jnp.min(jnp.where(d2 == m, j, n2), axis=1, keepdims=True)
        idxs.append(ik)
        invs.append(1.0 / jnp.maximum(m, 1e-10))
        if k < 2:
            d2 = jnp.where(j == ik, jnp.inf, d2)
    norm = invs[0] + invs[1] + invs[2]
    idx_ref[0] = jnp.concatenate([ik + b * n2 for ik in idxs], axis=1)
    w_ref[0] = jnp.concatenate(
        [jnp.broadcast_to(inv / norm, (blk, _L)) for inv in invs], axis=1)


def _mlp_kernel(it_ref, p1_ref, w0a_ref, w0b_ref, b0_ref, w1_ref, b1_ref,
                o_ref):
    h = jnp.dot(it_ref[...], w0a_ref[...], preferred_element_type=jnp.float32)
    h = h + jnp.dot(p1_ref[...], w0b_ref[...],
                    preferred_element_type=jnp.float32)
    h = jnp.maximum(h + b0_ref[...], 0.0)
    o = jnp.dot(h, w1_ref[...], preferred_element_type=jnp.float32)
    o_ref[...] = jnp.maximum(o + b1_ref[...], 0.0)


def kernel(xyz1, xyz2, points1, points2, W0, b0, W1, b1):
    B, N1, _ = xyz1.shape
    N2 = xyz2.shape[1]
    C1 = points1.shape[2]
    C2 = points2.shape[2]
    H = W0.shape[1]
    H2 = W1.shape[1]
    Q = N1

    BLK = 512
    nn3 = pl.pallas_call(
        functools.partial(_nn3_kernel, N2, BLK),
        grid=(1, N1 // BLK),
        in_specs=[
            pl.BlockSpec((1, BLK, 3), lambda b, n: (b, n, 0)),
            pl.BlockSpec((1, 3, N2), lambda b, n: (b, 0, 0)),
        ],
        out_specs=[
            pl.BlockSpec((1, BLK, 3), lambda b, n: (b, n, 0)),
            pl.BlockSpec((1, BLK, 3 * _L), lambda b, n: (b, n, 0)),
        ],
        out_shape=[
            jax.ShapeDtypeStruct((1, N1, 3), jnp.int32),
            jax.ShapeDtypeStruct((1, N1, 3 * _L), jnp.float32),
        ],
    )

    QPW = Q // _NW
    CH = 32
    NCH = QPW // CH
    nf = C2 // _L
    mesh = plsc.VectorSubcoreMesh(core_axis_name="c", subcore_axis_name="s")

    @functools.partial(
        pl.kernel,
        mesh=mesh,
        out_type=jax.ShapeDtypeStruct((Q, C2), jnp.float32),
        scratch_types=[
            pltpu.VMEM((QPW * 3,), jnp.int32),
            pltpu.VMEM((CH * 3 * _L,), jnp.float32),
            pltpu.VMEM((CH * 3 * _L,), jnp.float32),
            pltpu.VMEM((CH * 3, C2), jnp.float32),
            pltpu.VMEM((CH * 3, C2), jnp.float32),
            pltpu.VMEM((CH, C2), jnp.float32),
            pltpu.VMEM((CH, C2), jnp.float32),
            pltpu.SemaphoreType.DMA,
            pltpu.SemaphoreType.DMA,
            pltpu.SemaphoreType.DMA,
            pltpu.SemaphoreType.DMA,
            pltpu.SemaphoreType.DMA,
            pltpu.SemaphoreType.DMA,
        ],
    )
    def sc_interp(p2_hbm, idx_hbm, w_hbm, out_hbm, idx_v, w_b0, w_b1, r_b0,
                  r_b1, o_b0, o_b1, sg0, sg1, sw0, sw1, so0, so1):
        wid = lax.axis_index("s") * _NC + lax.axis_index("c")
        qw = wid * QPW
        w_b, r_b, o_b = [w_b0, w_b1], [r_b0, r_b1], [o_b0, o_b1]
        sg, sw, so = [sg0, sg1], [sw0, sw1], [so0, so1]
        gd, wd, od = [None, None], [None, None], [None, None]

        pltpu.sync_copy(idx_hbm.at[pl.ds(qw * 3, QPW * 3)], idx_v)

        def start(ci):
            buf = ci % 2
            gd[buf] = pltpu.async_copy(
                p2_hbm.at[idx_v.at[pl.ds(ci * CH * 3, CH * 3)]], r_b[buf],
                sg[buf])
            wd[buf] = pltpu.async_copy(
                w_hbm.at[pl.ds((qw + ci * CH) * 3 * _L, CH * 3 * _L)],
                w_b[buf], sw[buf])

        start(0)
        for ci in range(NCH):
            buf = ci % 2
            if ci + 1 < NCH:
                start(ci + 1)
            gd[buf].wait()
            wd[buf].wait()
            if od[buf] is not None:
                od[buf].wait()
            rows, wv, ov = r_b[buf], w_b[buf], o_b[buf]

            def body(i, _):
                for u in range(2):
                    q = 2 * i + u
                    base = q * 3 * _L
                    w0v = wv[pl.ds(base, _L)]
                    w1v = wv[pl.ds(base + _L, _L)]
                    w2v = wv[pl.ds(base + 2 * _L, _L)]
                    for f in range(nf):
                        sl = pl.ds(f * _L, _L)
                        acc = w0v * rows[3 * q, sl]
                        acc = acc + w1v * rows[3 * q + 1, sl]
                        acc = acc + w2v * rows[3 * q + 2, sl]
                        ov[q, sl] = acc
                return 0

            lax.fori_loop(0, CH // 2, body, 0)
            od[buf] = pltpu.async_copy(
                ov, out_hbm.at[pl.ds(qw + ci * CH, CH)], so[buf])
        od[0].wait()
        od[1].wait()

    MB = 1024
    mlp = pl.pallas_call(
        _mlp_kernel,
        grid=(Q // MB,),
        in_specs=[
            pl.BlockSpec((MB, C2), lambda r: (r, 0)),
            pl.BlockSpec((MB, C1), lambda r: (r, 0)),
            pl.BlockSpec((C2, H), lambda r: (0, 0)),
            pl.BlockSpec((C1, H), lambda r: (0, 0)),
            pl.BlockSpec((1, H), lambda r: (0, 0)),
            pl.BlockSpec((H, H2), lambda r: (0, 0)),
            pl.BlockSpec((1, H2), lambda r: (0, 0)),
        ],
        out_specs=pl.BlockSpec((MB, H2), lambda r: (r, 0)),
        out_shape=jax.ShapeDtypeStruct((Q, H2), jnp.float32),
    )

    xyz2t = xyz2.transpose(0, 2, 1)
    W0a, W0b = W0[:C2], W0[C2:]
    b0r, b1r = b0.reshape(1, H), b1.reshape(1, H2)
    outs = []
    for b in range(B):
        idx3, w3 = nn3(xyz1[b:b + 1], xyz2t[b:b + 1])
        interp = sc_interp(points2[b], idx3.reshape(Q * 3),
                           w3.reshape(Q * 3 * _L))
        outs.append(mlp(interp, points1[b], W0a, W0b, b0r, W1, b1r))
    return jnp.stack(outs)
